# Initial kernel scaffold; baseline (speedup 1.0000x reference)
import jax
import jax.numpy as jnp
from jax.experimental import pallas as pl


def kernel(x, edge_index, edge_weight, W):
    raise NotImplementedError("write your pallas kernel here")


# speedup vs baseline: 4.4767x; 4.4767x over previous
import functools

import jax
import jax.numpy as jnp
from jax import lax
from jax.experimental import pallas as pl
from jax.experimental.pallas import tpu as pltpu
from jax.experimental.pallas import tpu_sc as plsc

N = 10000
E = 320000
D = 128

NC = 2
NS = 16
NW = NC * NS
EDGES_PER_TILE = E // NW
CHUNK = 80
N_CHUNKS = EDGES_PER_TILE // CHUNK
NPAD = 10240
ROWS_PER_TILE = NPAD // NS


def _sc_aggregate(x, src, dst, edge_weight):
    mesh = plsc.VectorSubcoreMesh(core_axis_name="c", subcore_axis_name="s")

    @functools.partial(
        pl.kernel,
        out_type=jax.ShapeDtypeStruct((NC, NPAD, D), jnp.float32),
        mesh=mesh,
        scratch_types=[
            pltpu.VMEM((CHUNK,), jnp.int32),
            pltpu.VMEM((CHUNK,), jnp.int32),
            pltpu.VMEM((CHUNK,), jnp.float32),
            pltpu.VMEM((CHUNK, D), jnp.float32),
            pltpu.VMEM_SHARED((NPAD, D), jnp.float32),
        ],
    )
    def agg(x_hbm, src_hbm, dst_hbm, ew_hbm, out_hbm,
            src_v, dst_v, w_v, rows_v, acc):
        c = lax.axis_index("c")
        s = lax.axis_index("s")
        wid = c * NS + s

        zvec = jnp.zeros((16,), jnp.float32)

        def zero_row(r, _):
            for j in range(D // 16):
                rows_v[r, pl.ds(j * 16, 16)] = zvec
            return 0

        lax.fori_loop(0, CHUNK, zero_row, 0)
        row0 = s * ROWS_PER_TILE
        for k in range(ROWS_PER_TILE // CHUNK):
            pltpu.sync_copy(rows_v, acc.at[pl.ds(row0 + k * CHUNK, CHUNK)])
        plsc.subcore_barrier()

        base = wid * EDGES_PER_TILE

        def do_chunk(g, _):
            off = base + g * CHUNK
            pltpu.sync_copy(src_hbm.at[pl.ds(off, CHUNK)], src_v)
            pltpu.sync_copy(dst_hbm.at[pl.ds(off, CHUNK)], dst_v)
            pltpu.sync_copy(ew_hbm.at[pl.ds(off, CHUNK)], w_v)
            pltpu.sync_copy(x_hbm.at[src_v], rows_v)

            def scale_grp(k, _):
                wv = w_v[pl.ds(k * 16, 16)]
                for l in range(16):
                    wbc = jnp.full((16,), wv[l], jnp.float32)
                    e = k * 16 + l
                    for j in range(D // 16):
                        seg = rows_v[e, pl.ds(j * 16, 16)]
                        rows_v[e, pl.ds(j * 16, 16)] = seg * wbc
                return 0

            lax.fori_loop(0, CHUNK // 16, scale_grp, 0)

            pltpu.sync_copy(rows_v, acc.at[dst_v], add=True)
            return 0

        lax.fori_loop(0, N_CHUNKS, do_chunk, 0)
        plsc.subcore_barrier()

        pltpu.sync_copy(acc.at[pl.ds(row0, ROWS_PER_TILE)],
                        out_hbm.at[c, pl.ds(row0, ROWS_PER_TILE)])

    return agg(x, src, dst, edge_weight)


def _mm_body(p_ref, w_ref, o_ref):
    s = p_ref[0] + p_ref[1]
    o_ref[...] = jnp.dot(s, w_ref[...], preferred_element_type=jnp.float32)


BM = 400


def _tc_matmul(partials, W):
    return pl.pallas_call(
        _mm_body,
        grid=(N // BM,),
        in_specs=[
            pl.BlockSpec((NC, BM, D), lambda i: (0, i, 0)),

            pl.BlockSpec((D, D), lambda i: (0, 0)),
        ],
        out_specs=pl.BlockSpec((BM, D), lambda i: (i, 0)),
        out_shape=jax.ShapeDtypeStruct((N, D), jnp.float32),
    )(partials, W)


@jax.jit
def kernel(x, edge_index, edge_weight, W):
    partials = _sc_aggregate(x, edge_index[0], edge_index[1], edge_weight)
    return _tc_matmul(partials, W)


# speedup vs baseline: 10.9692x; 2.4503x over previous
import functools

import jax
import jax.numpy as jnp
from jax import lax
from jax.experimental import pallas as pl
from jax.experimental.pallas import tpu as pltpu
from jax.experimental.pallas import tpu_sc as plsc

N = 10000
E = 320000
D = 128

NC = 2
NS = 16
NW = NC * NS
EDGES_PER_TILE = E // NW
CHUNK = 80
N_CHUNKS = EDGES_PER_TILE // CHUNK
NPAD = 10240
ROWS_PER_TILE = NPAD // NS


def _sc_aggregate(x, src, dst, edge_weight):
    mesh = plsc.VectorSubcoreMesh(core_axis_name="c", subcore_axis_name="s")

    @functools.partial(
        pl.kernel,
        out_type=jax.ShapeDtypeStruct((NC, NPAD, D), jnp.float32),
        mesh=mesh,
        scratch_types=[
            pltpu.VMEM((4, CHUNK), jnp.int32),
            pltpu.VMEM((4, CHUNK), jnp.int32),
            pltpu.VMEM((4, CHUNK), jnp.float32),
            pltpu.VMEM((CHUNK, D), jnp.float32),
            pltpu.VMEM((CHUNK, D), jnp.float32),
            pltpu.VMEM_SHARED((NPAD, D), jnp.float32),
            [pltpu.SemaphoreType.DMA] * 4,
            pltpu.SemaphoreType.DMA,
            pltpu.SemaphoreType.DMA,
        ],
    )
    def agg(x_hbm, src_hbm, dst_hbm, ew_hbm, out_hbm,
            src_v, dst_v, w_v, rows0, rows1, acc, isems, gsem0, gsem1):
        c = lax.axis_index("c")
        s = lax.axis_index("s")
        wid = c * NS + s
        base = wid * EDGES_PER_TILE

        def stage_idx(g, slot):
            off = base + g * CHUNK
            pltpu.async_copy(src_hbm.at[pl.ds(off, CHUNK)],
                             src_v.at[slot], isems[slot])
            pltpu.async_copy(dst_hbm.at[pl.ds(off, CHUNK)],
                             dst_v.at[slot], isems[slot])
            pltpu.async_copy(ew_hbm.at[pl.ds(off, CHUNK)],
                             w_v.at[slot], isems[slot])

        def wait_idx(g, slot):
            off = base + g * CHUNK
            pltpu.make_async_copy(src_hbm.at[pl.ds(off, CHUNK)],
                                  src_v.at[slot], isems[slot]).wait()
            pltpu.make_async_copy(dst_hbm.at[pl.ds(off, CHUNK)],
                                  dst_v.at[slot], isems[slot]).wait()
            pltpu.make_async_copy(ew_hbm.at[pl.ds(off, CHUNK)],
                                  w_v.at[slot], isems[slot]).wait()

        for g in range(4):
            stage_idx(g, g)

        zvec = jnp.zeros((16,), jnp.float32)

        def zero_row(r, _):
            for j in range(D // 16):
                rows0[r, pl.ds(j * 16, 16)] = zvec
            return 0

        lax.fori_loop(0, CHUNK, zero_row, 0)
        row0 = s * ROWS_PER_TILE
        for k in range(ROWS_PER_TILE // CHUNK):
            pltpu.sync_copy(rows0, acc.at[pl.ds(row0 + k * CHUNK, CHUNK)])

        bufs = (rows0, rows1)
        sems = (gsem0, gsem1)

        def issue_gather(slot, rows, sem):
            pltpu.async_copy(x_hbm.at[src_v.at[slot]], rows, sem)

        def wait_gather(slot, rows, sem):
            pltpu.make_async_copy(x_hbm.at[src_v.at[slot]], rows, sem).wait()

        def scale(rows, slot):
            def grp(k16, _):
                wv = w_v[slot, pl.ds(k16 * 16, 16)]
                for l in range(16):
                    wbc = jnp.full((16,), wv[l], jnp.float32)
                    e = k16 * 16 + l
                    for j in range(D // 16):
                        seg = rows[e, pl.ds(j * 16, 16)]
                        rows[e, pl.ds(j * 16, 16)] = seg * wbc
                return 0

            lax.fori_loop(0, CHUNK // 16, grp, 0)

        wait_idx(0, 0)
        issue_gather(0, rows0, gsem0)
        wait_idx(1, 1)
        issue_gather(1, rows1, gsem1)

        plsc.subcore_barrier()

        def process(g, slot, rows, sem):
            wait_gather(slot, rows, sem)
            scale(rows, slot)
            pltpu.sync_copy(rows, acc.at[dst_v.at[slot]], add=True)

            @pl.when(g + 4 < N_CHUNKS)
            def _():
                stage_idx(g + 4, slot)

            @pl.when(g + 2 < N_CHUNKS)
            def _():
                wait_idx(g + 2, (slot + 2) % 4)
                issue_gather((slot + 2) % 4, rows, sem)

        def quad(k, _):
            for b in range(4):
                process(4 * k + b, b, bufs[b % 2], sems[b % 2])
            return 0

        lax.fori_loop(0, N_CHUNKS // 4, quad, 0)
        for b in range(N_CHUNKS % 4):
            g = (N_CHUNKS // 4) * 4 + b
            process(g, b, bufs[b % 2], sems[b % 2])

        plsc.subcore_barrier()

        pltpu.sync_copy(acc.at[pl.ds(row0, ROWS_PER_TILE)],
                        out_hbm.at[c, pl.ds(row0, ROWS_PER_TILE)])

    return agg(x, src, dst, edge_weight)


def _mm_body(p_ref, w_ref, o_ref):
    s = p_ref[0] + p_ref[1]
    o_ref[...] = jnp.dot(s, w_ref[...], preferred_element_type=jnp.float32)


BM = 400


def _tc_matmul(partials, W):
    return pl.pallas_call(
        _mm_body,
        grid=(N // BM,),
        in_specs=[
            pl.BlockSpec((NC, BM, D), lambda i: (0, i, 0)),
            pl.BlockSpec((D, D), lambda i: (0, 0)),
        ],
        out_specs=pl.BlockSpec((BM, D), lambda i: (i, 0)),
        out_shape=jax.ShapeDtypeStruct((N, D), jnp.float32),
    )(partials, W)


@jax.jit
def kernel(x, edge_index, edge_weight, W):
    partials = _sc_aggregate(x, edge_index[0], edge_index[1], edge_weight)
    return _tc_matmul(partials, W)


# speedup vs baseline: 12.6094x; 1.1495x over previous
import functools

import jax
import jax.numpy as jnp
from jax import lax
from jax.experimental import pallas as pl
from jax.experimental.pallas import tpu as pltpu
from jax.experimental.pallas import tpu_sc as plsc

N = 10000
E = 320000
D = 128

NC = 2
NS = 16
NW = NC * NS
EDGES_PER_TILE = E // NW
CHUNK = 80
N_CHUNKS = EDGES_PER_TILE // CHUNK
NB = 4
NI = 8
STRIPE = 624


def _sc_aggregate(x, src, dst, edge_weight):
    mesh = plsc.VectorSubcoreMesh(core_axis_name="c", subcore_axis_name="s")

    @functools.partial(
        pl.kernel,
        out_type=jax.ShapeDtypeStruct((NC, N, D), jnp.float32),
        mesh=mesh,
        scratch_types=[
            pltpu.VMEM((NI, CHUNK), jnp.int32),
            pltpu.VMEM((NI, CHUNK), jnp.int32),
            pltpu.VMEM((NI, CHUNK), jnp.float32),
            [pltpu.VMEM((CHUNK, D), jnp.float32)] * NB,
            pltpu.VMEM_SHARED((N, D), jnp.float32),
            [pltpu.SemaphoreType.DMA] * NI,
            [pltpu.SemaphoreType.DMA] * NB,
            [pltpu.SemaphoreType.DMA] * NB,
        ],
    )
    def agg(x_hbm, src_hbm, dst_hbm, ew_hbm, out_hbm,
            src_v, dst_v, w_v, bufs, acc, isems, gsems, ssems):
        c = lax.axis_index("c")
        s = lax.axis_index("s")
        wid = c * NS + s
        base = wid * EDGES_PER_TILE

        def stage_idx(g, slot):
            off = base + g * CHUNK
            pltpu.async_copy(src_hbm.at[pl.ds(off, CHUNK)],
                             src_v.at[slot], isems[slot])
            pltpu.async_copy(dst_hbm.at[pl.ds(off, CHUNK)],
                             dst_v.at[slot], isems[slot])
            pltpu.async_copy(ew_hbm.at[pl.ds(off, CHUNK)],
                             w_v.at[slot], isems[slot])

        def wait_idx(g, slot):
            off = base + g * CHUNK
            pltpu.make_async_copy(src_hbm.at[pl.ds(off, CHUNK)],
                                  src_v.at[slot], isems[slot]).wait()
            pltpu.make_async_copy(dst_hbm.at[pl.ds(off, CHUNK)],
                                  dst_v.at[slot], isems[slot]).wait()
            pltpu.make_async_copy(ew_hbm.at[pl.ds(off, CHUNK)],
                                  w_v.at[slot], isems[slot]).wait()

        def issue_gather(slot, b):
            pltpu.async_copy(x_hbm.at[src_v.at[slot]], bufs[b], gsems[b])

        def wait_gather(slot, b):
            pltpu.make_async_copy(
                x_hbm.at[src_v.at[slot]], bufs[b], gsems[b]).wait()

        def issue_scatter(slot, b):
            pltpu.async_copy(bufs[b], acc.at[dst_v.at[slot]], ssems[b],
                             add=True)

        def wait_scatter(slot, b):
            pltpu.make_async_copy(
                bufs[b], acc.at[dst_v.at[slot]], ssems[b]).wait()

        for g in range(NI - 1):
            stage_idx(g, g)

        zvec = jnp.zeros((16,), jnp.float32)

        def zero_row(r, _):
            for j in range(D // 16):
                bufs[0][r, pl.ds(j * 16, 16)] = zvec
            return 0

        lax.fori_loop(0, CHUNK, zero_row, 0)
        row0 = s * STRIPE
        for k in range(STRIPE // CHUNK):
            pltpu.sync_copy(bufs[0], acc.at[pl.ds(row0 + k * CHUNK, CHUNK)])
        nfull = (STRIPE // CHUNK) * CHUNK
        pltpu.sync_copy(bufs[0].at[pl.ds(0, STRIPE - nfull)],
                        acc.at[pl.ds(row0 + nfull, STRIPE - nfull)])

        @pl.when(s == NS - 1)
        def _():
            pltpu.sync_copy(bufs[0].at[pl.ds(0, N - NS * STRIPE)],
                            acc.at[pl.ds(NS * STRIPE, N - NS * STRIPE)])

        def scale(b, slot):
            def grp(k16, _):
                wv = w_v[slot, pl.ds(k16 * 16, 16)]
                for l in range(16):
                    wbc = jnp.full((16,), wv[l], jnp.float32)
                    e = k16 * 16 + l
                    for j in range(D // 16):
                        seg = bufs[b][e, pl.ds(j * 16, 16)]
                        bufs[b][e, pl.ds(j * 16, 16)] = seg * wbc
                return 0

            lax.fori_loop(0, CHUNK // 16, grp, 0)

        for g in range(NB - 1):
            wait_idx(g, g)
            issue_gather(g, g)

        plsc.subcore_barrier()

        def when(cond, fn):
            if isinstance(cond, bool):
                if cond:
                    fn()
            else:
                pl.when(cond)(fn)

        def process(g, bi, si):
            wait_gather(si, bi)
            scale(bi, si)

            when(g >= 1,
                 lambda: wait_scatter((si - 1) % NI, (bi - 1) % NB))
            when(g + NI - 1 < N_CHUNKS,
                 lambda: stage_idx(g + NI - 1, (si - 1) % NI))

            def prefetch():
                wait_idx(g + NB - 1, (si + NB - 1) % NI)
                issue_gather((si + NB - 1) % NI, (bi - 1) % NB)

            when(g + NB - 1 < N_CHUNKS, prefetch)
            issue_scatter(si, bi)

        def octet(k, _):
            for b in range(NI):
                g = NI * k + b
                process(g, b % NB, b)
            return 0

        lax.fori_loop(0, N_CHUNKS // NI, octet, 0)
        for b in range(N_CHUNKS % NI):
            g = (N_CHUNKS // NI) * NI + b
            process(g, g % NB, g % NI)

        wait_scatter((N_CHUNKS - 1) % NI, (N_CHUNKS - 1) % NB)

        plsc.subcore_barrier()

        pltpu.sync_copy(acc.at[pl.ds(row0, STRIPE)],
                        out_hbm.at[c, pl.ds(row0, STRIPE)])

        @pl.when(s == NS - 1)
        def _():
            pltpu.sync_copy(acc.at[pl.ds(NS * STRIPE, N - NS * STRIPE)],
                            out_hbm.at[c, pl.ds(NS * STRIPE,
                                                N - NS * STRIPE)])

    return agg(x, src, dst, edge_weight)


def _mm_body(p_ref, w_ref, o_ref):
    s = p_ref[0] + p_ref[1]
    o_ref[...] = jnp.dot(s, w_ref[...], preferred_element_type=jnp.float32)


BM = 400


def _tc_matmul(partials, W):
    return pl.pallas_call(
        _mm_body,
        grid=(N // BM,),
        in_specs=[
            pl.BlockSpec((NC, BM, D), lambda i: (0, i, 0)),
            pl.BlockSpec((D, D), lambda i: (0, 0)),
        ],
        out_specs=pl.BlockSpec((BM, D), lambda i: (i, 0)),
        out_shape=jax.ShapeDtypeStruct((N, D), jnp.float32),
    )(partials, W)


@jax.jit
def kernel(x, edge_index, edge_weight, W):
    partials = _sc_aggregate(x, edge_index[0], edge_index[1], edge_weight)
    return _tc_matmul(partials, W)
